# baseline (speedup 1.0000x reference)
import functools

import jax
import jax.numpy as jnp
from jax import lax
from jax.experimental import pallas as pl
from jax.experimental.pallas import tpu as pltpu
from jax.experimental.pallas import tpu_sc as plsc

_NC, _NS, _L = 2, 16, 16
_NW = _NC * _NS


def _rowsum_body(e_ref, i_ref, g_ref, o_ref):
    acc = e_ref[...] + i_ref[...] + g_ref[...]
    o_ref[...] = jnp.sum(acc, axis=0)


def _make_tc_rowsum(V, D, blk=16384):
    grid = (V + blk - 1) // blk
    s_len = grid * blk
    in_spec = pl.BlockSpec((D, blk), lambda i: (0, i))
    return pl.pallas_call(
        _rowsum_body,
        grid=(grid,),
        in_specs=[in_spec, in_spec, in_spec],
        out_specs=pl.BlockSpec((blk,), lambda i: (i,)),
        out_shape=jax.ShapeDtypeStruct((s_len,), jnp.float32),
    )


def _make_sc_gather(B, F, s_len):
    b_per_w = B // _NW
    G = 128
    C = G * F
    n_chunks = b_per_w // G

    mesh = plsc.VectorSubcoreMesh(core_axis_name="c", subcore_axis_name="s")

    @functools.partial(
        pl.kernel,
        out_type=jax.ShapeDtypeStruct((B,), jnp.float32),
        mesh=mesh,
        scratch_types=(
            [pltpu.VMEM((C,), jnp.int32) for _ in range(4)] +
            [pltpu.VMEM((C,), jnp.float32) for _ in range(4)] +
            [pltpu.VMEM((b_per_w,), jnp.float32)] +
            [pltpu.VMEM_SHARED((s_len,), jnp.float32)] +
            [pltpu.SemaphoreType.DMA for _ in range(9)]
        ),
        compiler_params=pltpu.CompilerParams(use_tc_tiling_on_sc=False,
                                             needs_layout_passes=False),
    )
    def body(idx_hbm, s_hbm, out_hbm,
             idx_v0, idx_v1, idx_v2, idx_v3,
             val_v0, val_v1, val_v2, val_v3, out_v, s_sh,
             isem0, isem1, isem2, isem3, gsem0, gsem1, gsem2, gsem3, ssem):
        wid = lax.axis_index("s") * _NC + lax.axis_index("c")
        b0w = wid * b_per_w
        idx_bufs = (idx_v0, idx_v1, idx_v2, idx_v3)
        val_bufs = (val_v0, val_v1, val_v2, val_v3)
        isems = (isem0, isem1, isem2, isem3)
        gsems = (gsem0, gsem1, gsem2, gsem3)
        lanes = lax.iota(jnp.int32, _L)

        def idx_copies(g):
            b0 = b0w + g * G
            return [(idx_hbm.at[pl.ds(f * B + b0, G)],
                     idx_bufs[g].at[pl.ds(f * G, G)], isems[g])
                    for f in range(F)]

        def gather(g):
            return (s_sh.at[idx_bufs[g]], val_bufs[g], gsems[g])

        for g in range(n_chunks):
            for src, dst, sem in idx_copies(g):
                pltpu.async_copy(src, dst, sem)
        per_tile = s_len // _NS
        sid = lax.axis_index("s")
        stage = (s_hbm.at[pl.ds(sid * per_tile, per_tile)],
                 s_sh.at[pl.ds(sid * per_tile, per_tile)], ssem)
        pltpu.async_copy(*stage)
        pltpu.make_async_copy(*stage).wait()
        plsc.subcore_barrier()
        for src, dst, sem in idx_copies(0):
            pltpu.make_async_copy(src, dst, sem).wait()
        pltpu.async_copy(*gather(0))
        for g in range(n_chunks):
            if g + 1 < n_chunks:
                for src, dst, sem in idx_copies(g + 1):
                    pltpu.make_async_copy(src, dst, sem).wait()
                pltpu.async_copy(*gather(g + 1))
            pltpu.make_async_copy(*gather(g)).wait()
            for o in range(G // _L):
                pos = o * _L + lanes
                acc = plsc.load_gather(val_bufs[g], [pos])
                for f in range(1, F):
                    acc = acc + plsc.load_gather(val_bufs[g], [pos + f * G])
                out_v[pl.ds(g * G + o * _L, _L)] = acc
        pltpu.sync_copy(out_v, out_hbm.at[pl.ds(b0w, b_per_w)])

    return body


def kernel(sparse_input, emb_table, i_emb_table, g_emb_table):
    B, F = sparse_input.shape
    V, D = emb_table.shape
    idx_fmajor = sparse_input.T.reshape(-1)
    s = _make_tc_rowsum(V, D)(emb_table.T, i_emb_table.T, g_emb_table.T)
    return _make_sc_gather(B, F, s.shape[0])(idx_fmajor, s)


# speedup vs baseline: 1.1359x; 1.1359x over previous
import functools

import jax
import jax.numpy as jnp
from jax import lax
from jax.experimental import pallas as pl
from jax.experimental.pallas import tpu as pltpu
from jax.experimental.pallas import tpu_sc as plsc

_NC, _NS, _L = 2, 16, 16
_NW = _NC * _NS


def _rowsum_body(e_ref, i_ref, g_ref, o_ref):
    acc = e_ref[...] + i_ref[...] + g_ref[...]
    o_ref[...] = jnp.sum(acc, axis=0)


def _make_tc_rowsum(V, D, blk=49152):
    grid = (V + blk - 1) // blk
    s_len = grid * blk
    in_spec = pl.BlockSpec((D, blk), lambda i: (0, i))
    return pl.pallas_call(
        _rowsum_body,
        grid=(grid,),
        in_specs=[in_spec, in_spec, in_spec],
        out_specs=pl.BlockSpec((blk,), lambda i: (i,)),
        out_shape=jax.ShapeDtypeStruct((s_len,), jnp.float32),
    )


def _make_sc_gather(B, F, s_len):
    b_per_w = B // _NW
    G = 128
    C = G * F
    n_chunks = b_per_w // G

    mesh = plsc.VectorSubcoreMesh(core_axis_name="c", subcore_axis_name="s")

    @functools.partial(
        pl.kernel,
        out_type=jax.ShapeDtypeStruct((B,), jnp.float32),
        mesh=mesh,
        scratch_types=(
            [pltpu.VMEM((C,), jnp.int32) for _ in range(4)] +
            [pltpu.VMEM((C,), jnp.float32) for _ in range(4)] +
            [pltpu.VMEM((b_per_w,), jnp.float32)] +
            [pltpu.VMEM_SHARED((s_len,), jnp.float32)] +
            [pltpu.SemaphoreType.DMA for _ in range(9)]
        ),
        compiler_params=pltpu.CompilerParams(use_tc_tiling_on_sc=False,
                                             needs_layout_passes=False),
    )
    def body(idx_hbm, s_hbm, out_hbm,
             idx_v0, idx_v1, idx_v2, idx_v3,
             val_v0, val_v1, val_v2, val_v3, out_v, s_sh,
             isem0, isem1, isem2, isem3, gsem0, gsem1, gsem2, gsem3, ssem):
        wid = lax.axis_index("s") * _NC + lax.axis_index("c")
        b0w = wid * b_per_w
        idx_bufs = (idx_v0, idx_v1, idx_v2, idx_v3)
        val_bufs = (val_v0, val_v1, val_v2, val_v3)
        isems = (isem0, isem1, isem2, isem3)
        gsems = (gsem0, gsem1, gsem2, gsem3)
        lanes = lax.iota(jnp.int32, _L)

        def idx_copies(g):
            b0 = b0w + g * G
            return [(idx_hbm.at[pl.ds(f * B + b0, G)],
                     idx_bufs[g].at[pl.ds(f * G, G)], isems[g])
                    for f in range(F)]

        def gather(g):
            return (s_sh.at[idx_bufs[g]], val_bufs[g], gsems[g])

        for g in range(n_chunks):
            for src, dst, sem in idx_copies(g):
                pltpu.async_copy(src, dst, sem)
        per_tile = s_len // _NS
        sid = lax.axis_index("s")
        stage = (s_hbm.at[pl.ds(sid * per_tile, per_tile)],
                 s_sh.at[pl.ds(sid * per_tile, per_tile)], ssem)
        pltpu.async_copy(*stage)
        pltpu.make_async_copy(*stage).wait()
        plsc.subcore_barrier()
        for src, dst, sem in idx_copies(0):
            pltpu.make_async_copy(src, dst, sem).wait()
        pltpu.async_copy(*gather(0))
        for g in range(n_chunks):
            if g + 1 < n_chunks:
                for src, dst, sem in idx_copies(g + 1):
                    pltpu.make_async_copy(src, dst, sem).wait()
                pltpu.async_copy(*gather(g + 1))
            pltpu.make_async_copy(*gather(g)).wait()
            for o in range(G // _L):
                pos = o * _L + lanes
                acc = plsc.load_gather(val_bufs[g], [pos])
                for f in range(1, F):
                    acc = acc + plsc.load_gather(val_bufs[g], [pos + f * G])
                out_v[pl.ds(g * G + o * _L, _L)] = acc
        pltpu.sync_copy(out_v, out_hbm.at[pl.ds(b0w, b_per_w)])

    return body


def kernel(sparse_input, emb_table, i_emb_table, g_emb_table):
    B, F = sparse_input.shape
    V, D = emb_table.shape
    idx_fmajor = sparse_input.T.reshape(-1)
    s = _make_tc_rowsum(V, D)(emb_table.T, i_emb_table.T, g_emb_table.T)
    return _make_sc_gather(B, F, s.shape[0])(idx_fmajor, s)


# speedup vs baseline: 1.1430x; 1.0063x over previous
import functools

import jax
import jax.numpy as jnp
from jax import lax
from jax.experimental import pallas as pl
from jax.experimental.pallas import tpu as pltpu
from jax.experimental.pallas import tpu_sc as plsc

_NC, _NS, _L = 2, 16, 16
_NW = _NC * _NS


def _rowsum_body(e_ref, i_ref, g_ref, o_ref):
    acc = e_ref[...] + i_ref[...] + g_ref[...]
    o_ref[...] = jnp.sum(acc, axis=0)


def _make_tc_rowsum(V, D, blk=32768):
    grid = (V + blk - 1) // blk
    s_len = grid * blk
    in_spec = pl.BlockSpec((D, blk), lambda i: (0, i))
    return pl.pallas_call(
        _rowsum_body,
        grid=(grid,),
        in_specs=[in_spec, in_spec, in_spec],
        out_specs=pl.BlockSpec((blk,), lambda i: (i,)),
        out_shape=jax.ShapeDtypeStruct((s_len,), jnp.float32),
    )


def _make_sc_gather(B, F, s_len):
    b_per_w = B // _NW
    G = 128
    C = G * F
    n_chunks = b_per_w // G

    mesh = plsc.VectorSubcoreMesh(core_axis_name="c", subcore_axis_name="s")

    @functools.partial(
        pl.kernel,
        out_type=jax.ShapeDtypeStruct((B,), jnp.float32),
        mesh=mesh,
        scratch_types=(
            [pltpu.VMEM((C,), jnp.int32) for _ in range(4)] +
            [pltpu.VMEM((C,), jnp.float32) for _ in range(4)] +
            [pltpu.VMEM((b_per_w,), jnp.float32)] +
            [pltpu.VMEM_SHARED((s_len,), jnp.float32)] +
            [pltpu.SemaphoreType.DMA for _ in range(9)]
        ),
        compiler_params=pltpu.CompilerParams(use_tc_tiling_on_sc=False,
                                             needs_layout_passes=False),
    )
    def body(idx_hbm, s_hbm, out_hbm,
             idx_v0, idx_v1, idx_v2, idx_v3,
             val_v0, val_v1, val_v2, val_v3, out_v, s_sh,
             isem0, isem1, isem2, isem3, gsem0, gsem1, gsem2, gsem3, ssem):
        wid = lax.axis_index("s") * _NC + lax.axis_index("c")
        b0w = wid * b_per_w
        idx_bufs = (idx_v0, idx_v1, idx_v2, idx_v3)
        val_bufs = (val_v0, val_v1, val_v2, val_v3)
        isems = (isem0, isem1, isem2, isem3)
        gsems = (gsem0, gsem1, gsem2, gsem3)
        lanes = lax.iota(jnp.int32, _L)

        def idx_copies(g):
            b0 = b0w + g * G
            return [(idx_hbm.at[pl.ds(f * B + b0, G)],
                     idx_bufs[g].at[pl.ds(f * G, G)], isems[g])
                    for f in range(F)]

        def gather(g):
            return (s_sh.at[idx_bufs[g]], val_bufs[g], gsems[g])

        for g in range(n_chunks):
            for src, dst, sem in idx_copies(g):
                pltpu.async_copy(src, dst, sem)
        per_tile = s_len // _NS
        sid = lax.axis_index("s")
        stage = (s_hbm.at[pl.ds(sid * per_tile, per_tile)],
                 s_sh.at[pl.ds(sid * per_tile, per_tile)], ssem)
        pltpu.async_copy(*stage)
        pltpu.make_async_copy(*stage).wait()
        plsc.subcore_barrier()
        for src, dst, sem in idx_copies(0):
            pltpu.make_async_copy(src, dst, sem).wait()
        pltpu.async_copy(*gather(0))
        for g in range(n_chunks):
            if g + 1 < n_chunks:
                for src, dst, sem in idx_copies(g + 1):
                    pltpu.make_async_copy(src, dst, sem).wait()
                pltpu.async_copy(*gather(g + 1))
            pltpu.make_async_copy(*gather(g)).wait()
            for o in range(G // _L):
                pos = o * _L + lanes
                acc = plsc.load_gather(val_bufs[g], [pos])
                for f in range(1, F):
                    acc = acc + plsc.load_gather(val_bufs[g], [pos + f * G])
                out_v[pl.ds(g * G + o * _L, _L)] = acc
        pltpu.sync_copy(out_v, out_hbm.at[pl.ds(b0w, b_per_w)])

    return body


def kernel(sparse_input, emb_table, i_emb_table, g_emb_table):
    B, F = sparse_input.shape
    V, D = emb_table.shape
    idx_fmajor = sparse_input.T.reshape(-1)
    s = _make_tc_rowsum(V, D)(emb_table.T, i_emb_table.T, g_emb_table.T)
    return _make_sc_gather(B, F, s.shape[0])(idx_fmajor, s)
